# Initial kernel scaffold; baseline (speedup 1.0000x reference)
import jax
import jax.numpy as jnp
from jax.experimental import pallas as pl


def kernel(x, Wg, W_gate, W_up):
    raise NotImplementedError("write your pallas kernel here")


# speedup vs baseline: 1.3289x; 1.3289x over previous
import functools

import jax
import jax.numpy as jnp
from jax.experimental import pallas as pl
from jax.experimental.pallas import tpu as pltpu

E = 8
TOP_K = 2
D_IN = 1024
D_OUT = 1024
S = 2048


def _router_body(x_ref, wg_ref, gates_ref):
    x = x_ref[...]
    wg = wg_ref[...]
    logits = jnp.dot(x, wg, preferred_element_type=jnp.float32)
    e_iota = jax.lax.broadcasted_iota(jnp.int32, logits.shape, 1)
    m1 = jnp.max(logits, axis=-1, keepdims=True)
    i1 = jnp.min(jnp.where(logits == m1, e_iota, E), axis=-1, keepdims=True)
    masked = jnp.where(e_iota == i1, -jnp.inf, logits)
    m2 = jnp.max(masked, axis=-1, keepdims=True)
    i2 = jnp.min(jnp.where(masked == m2, e_iota, E), axis=-1, keepdims=True)
    t = jnp.exp(m2 - m1)
    g1 = 1.0 / (1.0 + t)
    g2 = t / (1.0 + t)
    gates = jnp.where(e_iota == i1, g1, 0.0) + jnp.where(e_iota == i2, g2, 0.0)
    gates_ref[...] = gates


def _expert_body(gates_ref, x_ref, wg_ref, wu_ref, out_ref):
    e = pl.program_id(0)
    x = x_ref[...]
    hg = jnp.dot(x, wg_ref[0], preferred_element_type=jnp.float32)
    hu = jnp.dot(x, wu_ref[0], preferred_element_type=jnp.float32)
    h = (hg * jax.nn.sigmoid(hg)) * hu
    gates = gates_ref[...]
    e_iota = jax.lax.broadcasted_iota(jnp.int32, gates.shape, 1)
    g_e = jnp.sum(jnp.where(e_iota == e, gates, 0.0), axis=-1, keepdims=True)
    contrib = g_e * h

    @pl.when(e == 0)
    def _():
        out_ref[...] = contrib

    @pl.when(e > 0)
    def _():
        out_ref[...] += contrib


@jax.jit
def _moe(x2, Wg, W_gate, W_up):
    gates = pl.pallas_call(
        _router_body,
        out_shape=jax.ShapeDtypeStruct((S, E), jnp.float32),
    )(x2, Wg)

    out = pl.pallas_call(
        _expert_body,
        grid=(E,),
        in_specs=[
            pl.BlockSpec((S, E), lambda e: (0, 0)),
            pl.BlockSpec((S, D_IN), lambda e: (0, 0)),
            pl.BlockSpec((1, D_IN, D_OUT), lambda e: (e, 0, 0)),
            pl.BlockSpec((1, D_IN, D_OUT), lambda e: (e, 0, 0)),
        ],
        out_specs=pl.BlockSpec((S, D_OUT), lambda e: (0, 0)),
        out_shape=jax.ShapeDtypeStruct((S, D_OUT), jnp.float32),
    )(gates, x2, W_gate, W_up)
    return out


def kernel(x, Wg, W_gate, W_up):
    B = x.shape[0]
    x2 = x.reshape(B * S, D_IN)
    out = _moe(x2, Wg, W_gate, W_up)
    return out.reshape(B, S, D_OUT)
